# baseline (speedup 1.0000x reference)
import jax
import jax.numpy as jnp
from jax import lax
from jax.experimental import pallas as pl
from jax.experimental.pallas import tpu as pltpu
from jax.experimental.pallas import tpu_sc as plsc

H = 64
NE = 32
NA = 16
NV = NE * NA
TAB_REP = 32

NC = 2
NS = 16
NW = NC * NS

B = 16384
L = 50
B_ROWS = B * L
ROWS_PER_W = B_ROWS // NW
IDX_MINOR = 128
IDX_ROWS_PER_W = ROWS_PER_W // IDX_MINOR
CHUNK_IDX_ROWS = 4
CHUNK_ROWS = CHUNK_IDX_ROWS * IDX_MINOR
N_CHUNKS = ROWS_PER_W // CHUNK_ROWS

TRB = 128


BPB = 8
PREP_GRID = B // (BPB * TRB)


def _hot(mat):
    return jnp.dot(mat, precision=lax.Precision.HIGHEST,
                   preferred_element_type=jnp.float32)


def _prep_body(et_ref, at_ref, w_ref, b_ref, ent_ref, attr_ref, c_ref, idx_ref):
    i = pl.program_id(0)

    @pl.when(i == 0)
    def _():
        e2 = jnp.dot(et_ref[...], w_ref[0:H, :], precision=lax.Precision.HIGHEST,
                     preferred_element_type=jnp.float32)
        a2 = jnp.dot(at_ref[...], w_ref[H:2 * H, :], precision=lax.Precision.HIGHEST,
                     preferred_element_type=jnp.float32)
        r = lax.broadcasted_iota(jnp.int32, (NV, NE), 0) // NA
        c = lax.broadcasted_iota(jnp.int32, (NV, NE), 1)
        oh_e = (r == c).astype(jnp.float32)
        r2 = lax.broadcasted_iota(jnp.int32, (NV, NA), 0) % NA
        c2 = lax.broadcasted_iota(jnp.int32, (NV, NA), 1)
        oh_a = (r2 == c2).astype(jnp.float32)
        c_tab = (jnp.dot(oh_e, e2, precision=lax.Precision.HIGHEST,
                         preferred_element_type=jnp.float32)
                 + jnp.dot(oh_a, a2, precision=lax.Precision.HIGHEST,
                           preferred_element_type=jnp.float32)
                 + b_ref[...])
        for k in range(TAB_REP):
            c_ref[k * NV:(k + 1) * NV, :] = c_tab

    m = (ent_ref[...] * NA + attr_ref[...]).astype(jnp.float32)
    rt = lax.broadcasted_iota(jnp.int32, (L, L), 0)
    rs = lax.broadcasted_iota(jnp.int32, (L, L), 1)
    cs = lax.broadcasted_iota(jnp.int32, (TRB, TRB), 0)
    ct = lax.broadcasted_iota(jnp.int32, (TRB, TRB), 1)
    sels = []
    for hi in (0, 1):
        for dl in (0, 1):
            sel_r = ((rt % 2 == hi) & (rs % 2 == dl)
                     & (rs // 2 == rt // 2)).astype(jnp.float32)
            sel_c = ((ct % 2 == dl)
                     & (cs == hi * 64 + ct // 2)).astype(jnp.float32)
            sels.append((sel_r, sel_c))
    for t in range(BPB):
        x = m[t * TRB:(t + 1) * TRB, :]
        y = jnp.zeros((L, TRB), jnp.float32)
        for sel_r, sel_c in sels:
            rx = lax.dot_general(
                sel_r, x, (((1,), (1,)), ((), ())),
                precision=lax.Precision.HIGHEST,
                preferred_element_type=jnp.float32)
            y = y + jnp.dot(rx, sel_c, precision=lax.Precision.HIGHEST,
                            preferred_element_type=jnp.float32)
        grow = lax.broadcasted_iota(jnp.int32, (L, TRB), 0) + (i * BPB + t) * L
        rep = (grow // IDX_ROWS_PER_W) % TAB_REP
        idx_ref[t * L:(t + 1) * L, :] = y.astype(jnp.int32) + rep * NV


def _sc_body(c_hbm, idx_hbm, out_hbm, idx_v, rows_v, gsems, ssems):
    wid = lax.axis_index("s") * NC + lax.axis_index("c")
    idx_row0 = wid * IDX_ROWS_PER_W
    out_row0 = wid * ROWS_PER_W

    pltpu.sync_copy(idx_hbm.at[pl.ds(idx_row0, IDX_ROWS_PER_W)], idx_v)

    def fire_gathers(c, slot):
        for j in range(CHUNK_IDX_ROWS):
            pltpu.async_copy(
                c_hbm.at[idx_v.at[c * CHUNK_IDX_ROWS + j]],
                rows_v.at[slot, pl.ds(j * IDX_MINOR, IDX_MINOR)],
                gsems[slot])

    def drain_gathers(slot):
        for j in range(CHUNK_IDX_ROWS):
            pltpu.make_async_copy(
                c_hbm.at[idx_v.at[j]],
                rows_v.at[slot, pl.ds(j * IDX_MINOR, IDX_MINOR)],
                gsems[slot]).wait()

    def fire_scatter(c, slot):
        pltpu.async_copy(
            rows_v.at[slot],
            out_hbm.at[pl.ds(out_row0 + c * CHUNK_ROWS, CHUNK_ROWS)],
            ssems[slot])

    def drain_scatter(slot):
        pltpu.make_async_copy(
            rows_v.at[slot],
            out_hbm.at[pl.ds(out_row0, CHUNK_ROWS)],
            ssems[slot]).wait()

    NBODY = N_CHUNKS // 3 - 1
    fire_gathers(0, 0)
    fire_gathers(1, 1)
    fire_gathers(2, 2)

    @pl.loop(0, NBODY)
    def _(i):
        c0 = i * 3
        for s in range(3):
            drain_gathers(s)
            fire_scatter(c0 + s, s)
        for s in range(3):
            drain_scatter(s)
            fire_gathers(c0 + 3 + s, s)

    c0 = NBODY * 3
    for s in range(3):
        drain_gathers(s)
        fire_scatter(c0 + s, s)
    for s in range(2):
        drain_scatter(s)
        fire_gathers(c0 + 3 + s, s)
    drain_scatter(2)
    for s in range(2):
        drain_gathers(s)
        fire_scatter(c0 + 3 + s, s)
    drain_scatter(0)
    drain_scatter(1)


def _tr_body(x_ref, o_ref):
    for k in range(L // 2):
        o_ref[pl.ds(k * TRB, TRB), :] = x_ref[pl.ds(k * TRB, TRB), :].T


def kernel(ent, attr, entity_table, attr_table, W, b):
    ent32 = ent.astype(jnp.int32)
    attr32 = attr.astype(jnp.int32)
    b2 = b.reshape(1, H)

    c_tab, idx = pl.pallas_call(
        _prep_body,
        grid=(PREP_GRID,),
        in_specs=[pl.BlockSpec((NE, H), lambda i: (0, 0)),
                  pl.BlockSpec((NA, H), lambda i: (0, 0)),
                  pl.BlockSpec((2 * H, H), lambda i: (0, 0)),
                  pl.BlockSpec((1, H), lambda i: (0, 0)),
                  pl.BlockSpec((BPB * TRB, L), lambda i: (i, 0)),
                  pl.BlockSpec((BPB * TRB, L), lambda i: (i, 0))],
        out_specs=(pl.BlockSpec((TAB_REP * NV, H), lambda i: (0, 0)),
                   pl.BlockSpec((BPB * L, IDX_MINOR), lambda i: (i, 0))),
        out_shape=(
            jax.ShapeDtypeStruct((TAB_REP * NV, H), jnp.float32),
            jax.ShapeDtypeStruct((B_ROWS // IDX_MINOR, IDX_MINOR), jnp.int32),
        ),
    )(entity_table, attr_table, W, b2, ent32, attr32)

    mesh = plsc.VectorSubcoreMesh(core_axis_name="c", subcore_axis_name="s",
                                  num_cores=NC, num_subcores=NS)
    tok = pl.kernel(
        _sc_body,
        out_type=jax.ShapeDtypeStruct((B_ROWS, H), jnp.float32),
        mesh=mesh,
        compiler_params=pltpu.CompilerParams(use_tc_tiling_on_sc=False),
        scratch_types=[
            pltpu.VMEM((IDX_ROWS_PER_W, IDX_MINOR), jnp.int32),
            pltpu.VMEM((3, CHUNK_ROWS, H), jnp.float32),
            [pltpu.SemaphoreType.DMA] * 3,
            [pltpu.SemaphoreType.DMA] * 3,
        ],
    )(c_tab, idx)

    x = tok.reshape(B_ROWS // 2, 2 * H)
    out2d = pl.pallas_call(
        _tr_body,
        grid=(B // TRB,),
        in_specs=[pl.BlockSpec((TRB * L // 2, 2 * H), lambda i: (i, 0))],
        out_specs=pl.BlockSpec((L * H, TRB), lambda i: (0, i)),
        out_shape=jax.ShapeDtypeStruct((L * H, B), jnp.float32),
    )(x)
    return out2d.reshape(L, H, B).transpose(2, 0, 1)


# speedup vs baseline: 1.2009x; 1.2009x over previous
import jax
import jax.numpy as jnp
from jax import lax
from jax.experimental import pallas as pl
from jax.experimental.pallas import tpu as pltpu
from jax.experimental.pallas import tpu_sc as plsc

H = 64
NE = 32
NA = 16
NV = NE * NA
TAB_REP = 32

NC = 2
NS = 16
NW = NC * NS

B = 16384
L = 50
B_ROWS = B * L
ROWS_PER_W = B_ROWS // NW
IDX_MINOR = 128
IDX_ROWS_PER_W = ROWS_PER_W // IDX_MINOR
CHUNK_IDX_ROWS = 4
CHUNK_ROWS = CHUNK_IDX_ROWS * IDX_MINOR
N_CHUNKS = ROWS_PER_W // CHUNK_ROWS

TRB = 128
TRBLK = 4


BPB = 8
PREP_GRID = B // (BPB * TRB)


def _hot(mat):
    return jnp.dot(mat, precision=lax.Precision.HIGHEST,
                   preferred_element_type=jnp.float32)


def _prep_body(et_ref, at_ref, w_ref, b_ref, ent_ref, attr_ref, c_ref, idx_ref):
    i = pl.program_id(0)

    @pl.when(i == 0)
    def _():
        e2 = jnp.dot(et_ref[...], w_ref[0:H, :], precision=lax.Precision.HIGHEST,
                     preferred_element_type=jnp.float32)
        a2 = jnp.dot(at_ref[...], w_ref[H:2 * H, :], precision=lax.Precision.HIGHEST,
                     preferred_element_type=jnp.float32)
        r = lax.broadcasted_iota(jnp.int32, (NV, NE), 0) // NA
        c = lax.broadcasted_iota(jnp.int32, (NV, NE), 1)
        oh_e = (r == c).astype(jnp.float32)
        r2 = lax.broadcasted_iota(jnp.int32, (NV, NA), 0) % NA
        c2 = lax.broadcasted_iota(jnp.int32, (NV, NA), 1)
        oh_a = (r2 == c2).astype(jnp.float32)
        c_tab = (jnp.dot(oh_e, e2, precision=lax.Precision.HIGHEST,
                         preferred_element_type=jnp.float32)
                 + jnp.dot(oh_a, a2, precision=lax.Precision.HIGHEST,
                           preferred_element_type=jnp.float32)
                 + b_ref[...])
        for k in range(TAB_REP):
            c_ref[k * NV:(k + 1) * NV, :] = c_tab

    me = ent_ref[...].astype(jnp.float32)
    ma = attr_ref[...].astype(jnp.float32)
    rt = lax.broadcasted_iota(jnp.int32, (L, L), 0)
    rs = lax.broadcasted_iota(jnp.int32, (L, L), 1)
    cs = lax.broadcasted_iota(jnp.int32, (TRB, TRB), 0)
    ct = lax.broadcasted_iota(jnp.int32, (TRB, TRB), 1)
    sels = []
    for hi in (0, 1):
        for dl in (0, 1):
            sel_r = ((rt % 2 == hi) & (rs % 2 == dl)
                     & (rs // 2 == rt // 2)).astype(jnp.float32)
            sel_c = ((ct % 2 == dl)
                     & (cs == hi * 64 + ct // 2)).astype(jnp.float32)
            sels.append((sel_r, sel_c))
    def permute_block(x):
        y = jnp.zeros((L, TRB), jnp.float32)
        for sel_r, sel_c in sels:
            rx = lax.dot_general(sel_r, x, (((1,), (1,)), ((), ())),
                                 preferred_element_type=jnp.float32)
            y = y + jnp.dot(rx, sel_c, preferred_element_type=jnp.float32)
        return y

    for t in range(BPB):
        ye = permute_block(me[t * TRB:(t + 1) * TRB, :])
        ya = permute_block(ma[t * TRB:(t + 1) * TRB, :])
        grow = lax.broadcasted_iota(jnp.int32, (L, TRB), 0) + (i * BPB + t) * L
        rep = (grow // IDX_ROWS_PER_W) % TAB_REP
        idx_ref[t * L:(t + 1) * L, :] = (ye.astype(jnp.int32) * NA
                                         + ya.astype(jnp.int32) + rep * NV)


def _sc_body(c_hbm, idx_hbm, out_hbm, idx_v, rows_v, gsems, ssems):
    wid = lax.axis_index("s") * NC + lax.axis_index("c")
    idx_row0 = wid * IDX_ROWS_PER_W
    out_row0 = wid * ROWS_PER_W

    pltpu.sync_copy(idx_hbm.at[pl.ds(idx_row0, IDX_ROWS_PER_W)], idx_v)

    def fire_gathers(c, slot):
        for j in range(CHUNK_IDX_ROWS):
            pltpu.async_copy(
                c_hbm.at[idx_v.at[c * CHUNK_IDX_ROWS + j]],
                rows_v.at[slot, pl.ds(j * IDX_MINOR, IDX_MINOR)],
                gsems[slot])

    def drain_gathers(slot):
        for j in range(CHUNK_IDX_ROWS):
            pltpu.make_async_copy(
                c_hbm.at[idx_v.at[j]],
                rows_v.at[slot, pl.ds(j * IDX_MINOR, IDX_MINOR)],
                gsems[slot]).wait()

    def fire_scatter(c, slot):
        pltpu.async_copy(
            rows_v.at[slot],
            out_hbm.at[pl.ds(out_row0 + c * CHUNK_ROWS, CHUNK_ROWS)],
            ssems[slot])

    def drain_scatter(slot):
        pltpu.make_async_copy(
            rows_v.at[slot],
            out_hbm.at[pl.ds(out_row0, CHUNK_ROWS)],
            ssems[slot]).wait()

    NBODY = N_CHUNKS // 3 - 1
    fire_gathers(0, 0)
    fire_gathers(1, 1)
    fire_gathers(2, 2)

    @pl.loop(0, NBODY)
    def _(i):
        c0 = i * 3
        for s in range(3):
            drain_gathers(s)
            fire_scatter(c0 + s, s)
        for s in range(3):
            drain_scatter(s)
            fire_gathers(c0 + 3 + s, s)

    c0 = NBODY * 3
    for s in range(3):
        drain_gathers(s)
        fire_scatter(c0 + s, s)
    for s in range(2):
        drain_scatter(s)
        fire_gathers(c0 + 3 + s, s)
    drain_scatter(2)
    for s in range(2):
        drain_gathers(s)
        fire_scatter(c0 + 3 + s, s)
    drain_scatter(0)
    drain_scatter(1)


def _tr_body(x_ref, o_ref):
    for g in range(TRBLK):
        for k in range(L // 2):
            o_ref[pl.ds(k * TRB, TRB), pl.ds(g * TRB, TRB)] = (
                x_ref[pl.ds((g * (L // 2) + k) * TRB, TRB), :].T)


def kernel(ent, attr, entity_table, attr_table, W, b):
    ent32 = ent.astype(jnp.int32)
    attr32 = attr.astype(jnp.int32)
    b2 = b.reshape(1, H)

    c_tab, idx = pl.pallas_call(
        _prep_body,
        grid=(PREP_GRID,),
        in_specs=[pl.BlockSpec((NE, H), lambda i: (0, 0)),
                  pl.BlockSpec((NA, H), lambda i: (0, 0)),
                  pl.BlockSpec((2 * H, H), lambda i: (0, 0)),
                  pl.BlockSpec((1, H), lambda i: (0, 0)),
                  pl.BlockSpec((BPB * TRB, L), lambda i: (i, 0)),
                  pl.BlockSpec((BPB * TRB, L), lambda i: (i, 0))],
        out_specs=(pl.BlockSpec((TAB_REP * NV, H), lambda i: (0, 0)),
                   pl.BlockSpec((BPB * L, IDX_MINOR), lambda i: (i, 0))),
        out_shape=(
            jax.ShapeDtypeStruct((TAB_REP * NV, H), jnp.float32),
            jax.ShapeDtypeStruct((B_ROWS // IDX_MINOR, IDX_MINOR), jnp.int32),
        ),
    )(entity_table, attr_table, W, b2, ent32, attr32)

    mesh = plsc.VectorSubcoreMesh(core_axis_name="c", subcore_axis_name="s",
                                  num_cores=NC, num_subcores=NS)
    tok = pl.kernel(
        _sc_body,
        out_type=jax.ShapeDtypeStruct((B_ROWS, H), jnp.float32),
        mesh=mesh,
        compiler_params=pltpu.CompilerParams(use_tc_tiling_on_sc=False),
        scratch_types=[
            pltpu.VMEM((IDX_ROWS_PER_W, IDX_MINOR), jnp.int32),
            pltpu.VMEM((3, CHUNK_ROWS, H), jnp.float32),
            [pltpu.SemaphoreType.DMA] * 3,
            [pltpu.SemaphoreType.DMA] * 3,
        ],
    )(c_tab, idx)

    x = tok.reshape(B_ROWS // 2, 2 * H)
    out2d = pl.pallas_call(
        _tr_body,
        grid=(B // (TRB * TRBLK),),
        in_specs=[pl.BlockSpec((TRBLK * TRB * L // 2, 2 * H), lambda i: (i, 0))],
        out_specs=pl.BlockSpec((L * H, TRBLK * TRB), lambda i: (0, i)),
        out_shape=jax.ShapeDtypeStruct((L * H, B), jnp.float32),
    )(x)
    return out2d.reshape(L, H, B).transpose(2, 0, 1)


# speedup vs baseline: 1.3402x; 1.1160x over previous
import jax
import jax.numpy as jnp
from jax import lax
from jax.experimental import pallas as pl
from jax.experimental.pallas import tpu as pltpu
from jax.experimental.pallas import tpu_sc as plsc

H = 64
NE = 32
NA = 16
NV = NE * NA
TAB_REP = 32

NC = 2
NS = 16
NW = NC * NS

B = 16384
L = 50
B_ROWS = B * L
ROWS_PER_W = B_ROWS // NW
IDX_MINOR = 128
IDX_ROWS_PER_W = ROWS_PER_W // IDX_MINOR
CHUNK_IDX_ROWS = 4
CHUNK_ROWS = CHUNK_IDX_ROWS * IDX_MINOR
N_CHUNKS = ROWS_PER_W // CHUNK_ROWS

TRB = 128
TRBLK = 8


BPB = 8
PREP_GRID = B // (BPB * TRB)


def _hot(mat):
    return jnp.dot(mat, precision=lax.Precision.HIGHEST,
                   preferred_element_type=jnp.float32)


def _prep_body(et_ref, at_ref, w_ref, b_ref, ent_ref, attr_ref, c_ref, idx_ref):
    i = pl.program_id(0)

    @pl.when(i == 0)
    def _():
        e2 = jnp.dot(et_ref[...], w_ref[0:H, :], precision=lax.Precision.HIGHEST,
                     preferred_element_type=jnp.float32)
        a2 = jnp.dot(at_ref[...], w_ref[H:2 * H, :], precision=lax.Precision.HIGHEST,
                     preferred_element_type=jnp.float32)
        r = lax.broadcasted_iota(jnp.int32, (NV, NE), 0) // NA
        c = lax.broadcasted_iota(jnp.int32, (NV, NE), 1)
        oh_e = (r == c).astype(jnp.float32)
        r2 = lax.broadcasted_iota(jnp.int32, (NV, NA), 0) % NA
        c2 = lax.broadcasted_iota(jnp.int32, (NV, NA), 1)
        oh_a = (r2 == c2).astype(jnp.float32)
        c_tab = (jnp.dot(oh_e, e2, precision=lax.Precision.HIGHEST,
                         preferred_element_type=jnp.float32)
                 + jnp.dot(oh_a, a2, precision=lax.Precision.HIGHEST,
                           preferred_element_type=jnp.float32)
                 + b_ref[...])
        for k in range(TAB_REP):
            c_ref[k * NV:(k + 1) * NV, :] = c_tab

    me = ent_ref[...].astype(jnp.float32)
    ma = attr_ref[...].astype(jnp.float32)
    rt = lax.broadcasted_iota(jnp.int32, (L, L), 0)
    rs = lax.broadcasted_iota(jnp.int32, (L, L), 1)
    cs = lax.broadcasted_iota(jnp.int32, (TRB, TRB), 0)
    ct = lax.broadcasted_iota(jnp.int32, (TRB, TRB), 1)
    sel_rs, sel_cs = [], []
    for hi in (0, 1):
        for dl in (0, 1):
            sel_rs.append(((rt % 2 == hi) & (rs % 2 == dl)
                           & (rs // 2 == rt // 2)).astype(jnp.float32))
            sel_cs.append(((ct % 2 == dl)
                           & (cs == hi * 64 + ct // 2)).astype(jnp.float32))
    sel_r_cat = jnp.concatenate(sel_rs, axis=0)
    sel_c_cat = jnp.concatenate(sel_cs, axis=0)

    rxe = lax.dot_general(sel_r_cat, me, (((1,), (1,)), ((), ())),
                          preferred_element_type=jnp.float32)
    rxa = lax.dot_general(sel_r_cat, ma, (((1,), (1,)), ((), ())),
                          preferred_element_type=jnp.float32)

    def permute_block(rx, t):
        s = rx[:, t * TRB:(t + 1) * TRB]
        wide = jnp.concatenate([s[a * L:(a + 1) * L, :] for a in range(4)],
                               axis=1)
        return jnp.dot(wide, sel_c_cat, preferred_element_type=jnp.float32)

    for t in range(BPB):
        ye = permute_block(rxe, t)
        ya = permute_block(rxa, t)
        grow = lax.broadcasted_iota(jnp.int32, (L, TRB), 0) + (i * BPB + t) * L
        rep = (grow // IDX_ROWS_PER_W) % TAB_REP
        idx_ref[t * L:(t + 1) * L, :] = (ye.astype(jnp.int32) * NA
                                         + ya.astype(jnp.int32) + rep * NV)


def _sc_body(c_hbm, idx_hbm, out_hbm, idx_v, rows_v, gsems, ssems):
    wid = lax.axis_index("s") * NC + lax.axis_index("c")
    idx_row0 = wid * IDX_ROWS_PER_W
    out_row0 = wid * ROWS_PER_W

    pltpu.sync_copy(idx_hbm.at[pl.ds(idx_row0, IDX_ROWS_PER_W)], idx_v)

    def fire_gathers(c, slot):
        for j in range(CHUNK_IDX_ROWS):
            pltpu.async_copy(
                c_hbm.at[idx_v.at[c * CHUNK_IDX_ROWS + j]],
                rows_v.at[slot, pl.ds(j * IDX_MINOR, IDX_MINOR)],
                gsems[slot])

    def drain_gathers(slot):
        for j in range(CHUNK_IDX_ROWS):
            pltpu.make_async_copy(
                c_hbm.at[idx_v.at[j]],
                rows_v.at[slot, pl.ds(j * IDX_MINOR, IDX_MINOR)],
                gsems[slot]).wait()

    def fire_scatter(c, slot):
        pltpu.async_copy(
            rows_v.at[slot],
            out_hbm.at[pl.ds(out_row0 + c * CHUNK_ROWS, CHUNK_ROWS)],
            ssems[slot])

    def drain_scatter(slot):
        pltpu.make_async_copy(
            rows_v.at[slot],
            out_hbm.at[pl.ds(out_row0, CHUNK_ROWS)],
            ssems[slot]).wait()

    NBODY = N_CHUNKS // 3 - 1
    fire_gathers(0, 0)
    fire_gathers(1, 1)
    fire_gathers(2, 2)

    @pl.loop(0, NBODY)
    def _(i):
        c0 = i * 3
        for s in range(3):
            drain_gathers(s)
            fire_scatter(c0 + s, s)
        for s in range(3):
            drain_scatter(s)
            fire_gathers(c0 + 3 + s, s)

    c0 = NBODY * 3
    for s in range(3):
        drain_gathers(s)
        fire_scatter(c0 + s, s)
    for s in range(2):
        drain_scatter(s)
        fire_gathers(c0 + 3 + s, s)
    drain_scatter(2)
    for s in range(2):
        drain_gathers(s)
        fire_scatter(c0 + 3 + s, s)
    drain_scatter(0)
    drain_scatter(1)


def _tr_body(x_ref, o_ref):
    for g in range(TRBLK):
        for k in range(L // 2):
            o_ref[pl.ds(k * TRB, TRB), pl.ds(g * TRB, TRB)] = (
                x_ref[pl.ds((g * (L // 2) + k) * TRB, TRB), :].T)


def kernel(ent, attr, entity_table, attr_table, W, b):
    ent32 = ent.astype(jnp.int32)
    attr32 = attr.astype(jnp.int32)
    b2 = b.reshape(1, H)

    c_tab, idx = pl.pallas_call(
        _prep_body,
        grid=(PREP_GRID,),
        in_specs=[pl.BlockSpec((NE, H), lambda i: (0, 0)),
                  pl.BlockSpec((NA, H), lambda i: (0, 0)),
                  pl.BlockSpec((2 * H, H), lambda i: (0, 0)),
                  pl.BlockSpec((1, H), lambda i: (0, 0)),
                  pl.BlockSpec((BPB * TRB, L), lambda i: (i, 0)),
                  pl.BlockSpec((BPB * TRB, L), lambda i: (i, 0))],
        out_specs=(pl.BlockSpec((TAB_REP * NV, H), lambda i: (0, 0)),
                   pl.BlockSpec((BPB * L, IDX_MINOR), lambda i: (i, 0))),
        out_shape=(
            jax.ShapeDtypeStruct((TAB_REP * NV, H), jnp.float32),
            jax.ShapeDtypeStruct((B_ROWS // IDX_MINOR, IDX_MINOR), jnp.int32),
        ),
    )(entity_table, attr_table, W, b2, ent32, attr32)

    mesh = plsc.VectorSubcoreMesh(core_axis_name="c", subcore_axis_name="s",
                                  num_cores=NC, num_subcores=NS)
    tok = pl.kernel(
        _sc_body,
        out_type=jax.ShapeDtypeStruct((B_ROWS, H), jnp.float32),
        mesh=mesh,
        compiler_params=pltpu.CompilerParams(use_tc_tiling_on_sc=False),
        scratch_types=[
            pltpu.VMEM((IDX_ROWS_PER_W, IDX_MINOR), jnp.int32),
            pltpu.VMEM((3, CHUNK_ROWS, H), jnp.float32),
            [pltpu.SemaphoreType.DMA] * 3,
            [pltpu.SemaphoreType.DMA] * 3,
        ],
    )(c_tab, idx)

    x = tok.reshape(B_ROWS // 2, 2 * H)
    out2d = pl.pallas_call(
        _tr_body,
        grid=(B // (TRB * TRBLK),),
        in_specs=[pl.BlockSpec((TRBLK * TRB * L // 2, 2 * H), lambda i: (i, 0))],
        out_specs=pl.BlockSpec((L * H, TRBLK * TRB), lambda i: (0, i)),
        out_shape=jax.ShapeDtypeStruct((L * H, B), jnp.float32),
    )(x)
    return out2d.reshape(L, H, B).transpose(2, 0, 1)
